# baseline (speedup 1.0000x reference)
import jax
import jax.numpy as jnp
from jax.experimental import pallas as pl
from jax.experimental.pallas import tpu as pltpu

N_CAM = 8
SEG = 12500
SUB = 2048
B = 64
D = 128
BETA = 0.05

_OFFS = [(o, min(SUB, SEG - o)) for o in range(0, SEG, SUB)]


def _loss_kernel(x_ref, em_ref, lab_ref, out_ref):
    x = x_ref[...]
    xn = x / jnp.maximum(
        jnp.sqrt(jnp.sum(x * x, axis=1, keepdims=True)), 1e-12)

    m = jnp.full((B, 1), -1e30, jnp.float32)
    s = jnp.zeros((B, 1), jnp.float32)
    dotacc = jnp.zeros((B, D), jnp.float32)
    lsum = jnp.zeros((B, 1), jnp.float32)
    lmax = jnp.full((B, 1), -1e30, jnp.float32)

    for off, sz in _OFFS:
        em = em_ref[0, off:off + sz, :]
        lab = lab_ref[:, 0, 0, off:off + sz]

        logits = jax.lax.dot_general(
            xn, em, (((1,), (1,)), ((), ())),
            preferred_element_type=jnp.float32) * (1.0 / BETA)

        bm = jnp.max(logits, axis=1, keepdims=True)
        m_new = jnp.maximum(m, bm)
        s = (s * jnp.exp(m - m_new)
             + jnp.sum(jnp.exp(logits - m_new), axis=1, keepdims=True))
        m = m_new

        dotacc = dotacc + jnp.dot(lab, em, preferred_element_type=jnp.float32)
        lsum = lsum + jnp.sum(lab, axis=1, keepdims=True)
        lmax = jnp.maximum(lmax, jnp.max(lab, axis=1, keepdims=True))

    lse = m + jnp.log(s)
    rowdot = jnp.sum(xn * dotacc, axis=1,
                     keepdims=True) * (1.0 / BETA)
    v = (lsum * lse - rowdot) / (lmax + 1e-20)
    out_ref[0] = jnp.sum(v, axis=0, keepdims=True) / (B * N_CAM)


def kernel(x, pids, img_index, cams, labels, em_all):
    em_r = em_all.reshape(N_CAM, SEG, D)
    lab_r = labels.reshape(B, N_CAM, 1, SEG)

    out = pl.pallas_call(
        _loss_kernel,
        grid=(N_CAM,),
        in_specs=[
            pl.BlockSpec((B, D), lambda c: (0, 0)),
            pl.BlockSpec((1, SEG, D), lambda c: (c, 0, 0)),
            pl.BlockSpec((B, 1, 1, SEG), lambda c: (0, c, 0, 0)),
        ],
        out_specs=pl.BlockSpec((1, 1, 1), lambda c: (c, 0, 0)),
        out_shape=jax.ShapeDtypeStruct((N_CAM, 1, 1), jnp.float32),
        compiler_params=pltpu.CompilerParams(
            dimension_semantics=("parallel",)),
    )(x, em_r, lab_r)
    return jnp.sum(out).reshape(())
